# baseline (speedup 1.0000x reference)
import jax
import jax.numpy as jnp
from jax import lax
from jax.experimental import pallas as pl
from jax.experimental.pallas import tpu as pltpu
from jax.experimental.pallas import tpu_sc as plsc

_B = 200
_NW = 32


def _gather_body(table_hbm, idx_hbm, out_hbm, idx_v, rows_v, sem):
    b_per_w = idx_hbm.shape[0] // _NW
    wid = lax.axis_index("s") * 2 + lax.axis_index("c")
    base = wid * b_per_w
    pltpu.sync_copy(idx_hbm.at[pl.ds(base, b_per_w)], idx_v)
    pltpu.make_async_copy(table_hbm.at[idx_v], rows_v, sem).start()
    pltpu.make_async_copy(table_hbm.at[idx_v], rows_v, sem).wait()
    pltpu.sync_copy(rows_v, out_hbm.at[pl.ds(base, b_per_w)])


def _sc_gather(table, obj_ids):
    N, E = table.shape
    Bp = ((N - 1) // (8 * _NW) + 1) * (8 * _NW)
    idx = jnp.pad(obj_ids, (0, Bp - N))
    b_per_w = Bp // _NW
    mesh = plsc.VectorSubcoreMesh(core_axis_name="c", subcore_axis_name="s")
    emb = pl.kernel(
        _gather_body,
        out_type=jax.ShapeDtypeStruct((Bp, E), table.dtype),
        mesh=mesh,
        scratch_types=[
            pltpu.VMEM((b_per_w,), jnp.int32),
            pltpu.VMEM((b_per_w, E), table.dtype),
            pltpu.SemaphoreType.DMA,
        ],
        compiler_params=pltpu.CompilerParams(use_tc_tiling_on_sc=False),
    )(table, idx)
    return emb[:N]


def _interleave_kernel(states_ref, emb_ref, out_ref):
    e = emb_ref[...][:, 0, :]
    B = states_ref.shape[1]
    T = out_ref.shape[2]
    s = states_ref[0].reshape(B, T, 128)
    out_ref[0, :, :, :128] = s
    out_ref[0, :, :, 128:] = jnp.broadcast_to(e[:, None, :], (B, T, e.shape[-1]))


def kernel(states, table, obj_ids):
    Bt, N, T, D = states.shape
    E = table.shape[-1]
    emb = _sc_gather(table, obj_ids)
    flat = states.reshape(Bt, N, T * D)
    return pl.pallas_call(
        _interleave_kernel,
        grid=(Bt, N // _B),
        in_specs=[
            pl.BlockSpec((1, _B, T * D), lambda i, j: (i, j, 0)),
            pl.BlockSpec((_B, 1, E), lambda i, j: (j, 0, 0)),
        ],
        out_specs=pl.BlockSpec((1, _B, T, D + E), lambda i, j: (i, j, 0, 0)),
        out_shape=jax.ShapeDtypeStruct((Bt, N, T, D + E), states.dtype),
        compiler_params=pltpu.CompilerParams(
            dimension_semantics=("parallel", "parallel"),
            vmem_limit_bytes=100_000_000),
    )(flat, emb.reshape(N, 1, E))


# speedup vs baseline: 1.2926x; 1.2926x over previous
import jax
import jax.numpy as jnp
from jax import lax
from jax.experimental import pallas as pl
from jax.experimental.pallas import tpu as pltpu
from jax.experimental.pallas import tpu_sc as plsc

_B = 1000
_NW = 32


def _gather_body(table_hbm, idx_hbm, out_hbm, idx_v, rows_v, sem):
    b_per_w = idx_hbm.shape[0] // _NW
    wid = lax.axis_index("s") * 2 + lax.axis_index("c")
    base = wid * b_per_w
    pltpu.sync_copy(idx_hbm.at[pl.ds(base, b_per_w)], idx_v)
    pltpu.make_async_copy(table_hbm.at[idx_v], rows_v, sem).start()
    pltpu.make_async_copy(table_hbm.at[idx_v], rows_v, sem).wait()
    pltpu.sync_copy(rows_v, out_hbm.at[pl.ds(base, b_per_w)])


def _sc_gather(table, obj_ids):
    N, E = table.shape
    Bp = ((N - 1) // (8 * _NW) + 1) * (8 * _NW)
    idx = jnp.pad(obj_ids, (0, Bp - N))
    b_per_w = Bp // _NW
    mesh = plsc.VectorSubcoreMesh(core_axis_name="c", subcore_axis_name="s")
    emb = pl.kernel(
        _gather_body,
        out_type=jax.ShapeDtypeStruct((Bp, E), table.dtype),
        mesh=mesh,
        scratch_types=[
            pltpu.VMEM((b_per_w,), jnp.int32),
            pltpu.VMEM((b_per_w, E), table.dtype),
            pltpu.SemaphoreType.DMA,
        ],
        compiler_params=pltpu.CompilerParams(use_tc_tiling_on_sc=False),
    )(table, idx)
    return emb[:N]


def _interleave_kernel(states_ref, emb_ref, out_ref):
    e = emb_ref[...][:, 0, :]
    T = states_ref.shape[-1] // 128
    for k in range(T):
        out_ref[0, :, k * 160:k * 160 + 128] = states_ref[0, :, k * 128:(k + 1) * 128]
        out_ref[0, :, k * 160 + 128:(k + 1) * 160] = e


def kernel(states, table, obj_ids):
    Bt, N, T, D = states.shape
    E = table.shape[-1]
    emb = _sc_gather(table, obj_ids)
    flat = states.reshape(Bt, N, T * D)
    out = pl.pallas_call(
        _interleave_kernel,
        grid=(Bt, N // _B),
        in_specs=[
            pl.BlockSpec((1, _B, T * D), lambda i, j: (i, j, 0)),
            pl.BlockSpec((_B, 1, E), lambda i, j: (j, 0, 0)),
        ],
        out_specs=pl.BlockSpec((1, _B, T * (D + E)), lambda i, j: (i, j, 0)),
        out_shape=jax.ShapeDtypeStruct((Bt, N, T * (D + E)), states.dtype),
        compiler_params=pltpu.CompilerParams(
            dimension_semantics=("parallel", "parallel"),
            vmem_limit_bytes=100_000_000),
    )(flat, emb.reshape(N, 1, E))
    return out.reshape(Bt, N, T, D + E)
